# baseline (device time: 202984 ns/iter reference)
import functools

import jax
import jax.numpy as jnp
from jax import lax
from jax.experimental import pallas as pl
from jax.experimental.pallas import tpu as pltpu

N_DEV = 4
SQ = 2048
D = 1024
H_PER = 8
DH = 128
QB = 256
KB = 512
N_QB = SQ // QB
WINDOW = 128
SCALE = 0.08838834764831843


def kernel(x, Wq, K_ext, V_ext, Wo):
    x_bf = x.astype(jnp.bfloat16)
    wq_bf = Wq.astype(jnp.bfloat16)
    wo_bf = Wo.astype(jnp.bfloat16)

    def body(x_ref, wq_ref, wo_ref, k_hbm, v_hbm, out_ref,
             w_buf, q_buf, ctx_buf, k_stage, v_stage, kh_bf, vh_bf, bias_ref,
             kv_sems, ring_sems):
        my = lax.axis_index("i")
        left = (my + N_DEV - 1) % N_DEV
        right = (my + 1) % N_DEV

        barrier_sem = pltpu.get_barrier_semaphore()
        for nbr in (left, right):
            pl.semaphore_signal(
                barrier_sem, inc=1,
                device_id=(nbr,), device_id_type=pl.DeviceIdType.MESH,
            )
        pl.semaphore_wait(barrier_sem, 2)

        w_buf[0, :D, :] = wq_ref[...]
        w_buf[0, D:, :] = wo_ref[...]

        for idx in range(3):
            off = idx * WINDOW
            qi = off + lax.broadcasted_iota(jnp.int32, (QB, KB), 0)
            ki = lax.broadcasted_iota(jnp.int32, (QB, KB), 1)
            bias_ref[idx] = jnp.where(
                jnp.abs(qi - ki) <= WINDOW,
                jnp.float32(0.0), jnp.float32(-1e9),
            )

        def kv_copies(g, slot):
            ck = pltpu.make_async_copy(
                k_hbm.at[my, :, g, :], k_stage.at[slot], kv_sems.at[slot, 0]
            )
            cv = pltpu.make_async_copy(
                v_hbm.at[my, :, g, :], v_stage.at[slot], kv_sems.at[slot, 1]
            )
            return ck, cv

        def compute_stage(slot, origin, first):
            g0 = origin * H_PER

            cops = {0: kv_copies(g0, 0)}
            cops[0][0].start()
            cops[0][1].start()

            q = lax.dot_general(
                x_ref[0], w_buf[slot, :D, :],
                (((1,), (0,)), ((), ())),
                preferred_element_type=jnp.float32,
            )
            q_buf[...] = (q * SCALE).astype(jnp.bfloat16)

            for h in range(H_PER):
                kv_slot = h % 2
                if h + 1 < H_PER:
                    cops[h + 1] = kv_copies(g0 + h + 1, (h + 1) % 2)
                    cops[h + 1][0].start()
                    cops[h + 1][1].start()
                cops[h][0].wait()
                cops[h][1].wait()
                kh_bf[...] = k_stage[kv_slot].astype(jnp.bfloat16)
                vh_bf[...] = v_stage[kv_slot].astype(jnp.bfloat16)

                def attn_block(q0, s, bias, h=h):
                    q_blk = q_buf[pl.ds(q0, QB), h * DH:(h + 1) * DH]
                    k_blk = kh_bf[pl.ds(s, KB), :]
                    v_blk = vh_bf[pl.ds(s, KB), :]
                    sc = lax.dot_general(
                        q_blk, k_blk, (((1,), (1,)), ((), ())),
                        preferred_element_type=jnp.float32,
                    )
                    w = jnp.exp(sc + bias)
                    denom = jnp.sum(w, axis=1, keepdims=True)
                    wb = w.astype(jnp.bfloat16)
                    ctx = lax.dot_general(
                        wb, v_blk, (((1,), (0,)), ((), ())),
                        preferred_element_type=jnp.float32,
                    )
                    ctx_buf[pl.ds(q0, QB), h * DH:(h + 1) * DH] = (
                        (ctx * (1.0 / denom)).astype(jnp.bfloat16)
                    )

                attn_block(0, 0, bias_ref[0])

                def qb_body(qb, _):
                    q0 = pl.multiple_of(qb * QB, QB)
                    s = pl.multiple_of(q0 - WINDOW, WINDOW)
                    attn_block(q0, s, bias_ref[1])
                    return 0

                lax.fori_loop(1, N_QB - 1, qb_body, 0)
                attn_block(SQ - QB, SQ - KB, bias_ref[2])

            partial = lax.dot_general(
                ctx_buf[...], w_buf[slot, D:, :],
                (((1,), (0,)), ((), ())),
                preferred_element_type=jnp.float32,
            )
            if first:
                out_ref[0] = partial
            else:
                out_ref[0] = out_ref[0] + partial

        def ring_copy(src, dst, sem_i, target):
            return pltpu.make_async_remote_copy(
                src_ref=src, dst_ref=dst,
                send_sem=ring_sems.at[sem_i], recv_sem=ring_sems.at[sem_i + 1],
                device_id=(target,), device_id_type=pl.DeviceIdType.MESH,
            )

        a_r = ring_copy(w_buf.at[0], w_buf.at[1], 0, right)
        a_l = ring_copy(w_buf.at[0], w_buf.at[2], 2, left)
        a_r.start()
        a_l.start()
        compute_stage(0, my, True)
        a_r.wait()
        b_r = ring_copy(
            w_buf.at[1, pl.ds(0, D)], w_buf.at[3, pl.ds(0, D)], 4, right
        )
        b_r.start()
        compute_stage(1, (my + N_DEV - 1) % N_DEV, False)
        a_l.wait()
        b_l = ring_copy(
            w_buf.at[2, pl.ds(D, D)], w_buf.at[3, pl.ds(D, D)], 6, left
        )
        b_l.start()
        compute_stage(2, (my + 1) % N_DEV, False)
        b_r.wait()
        b_l.wait()
        compute_stage(3, (my + 2) % N_DEV, False)

        @functools.partial(
            pl.run_scoped, second_barrier=pltpu.SemaphoreType.REGULAR
        )
        def _(second_barrier):
            for nbr in (left, right):
                pl.semaphore_signal(
                    second_barrier, inc=1,
                    device_id=(nbr,), device_id_type=pl.DeviceIdType.MESH,
                )
            pl.semaphore_wait(second_barrier, 2)

    return pl.pallas_call(
        body,
        out_shape=jax.ShapeDtypeStruct((1, SQ, D), jnp.float32),
        in_specs=[
            pl.BlockSpec(memory_space=pltpu.VMEM),
            pl.BlockSpec(memory_space=pltpu.VMEM),
            pl.BlockSpec(memory_space=pltpu.VMEM),
            pl.BlockSpec(memory_space=pl.ANY),
            pl.BlockSpec(memory_space=pl.ANY),
        ],
        out_specs=pl.BlockSpec(memory_space=pltpu.VMEM),
        scratch_shapes=[
            pltpu.VMEM((N_DEV, 2 * D, D), jnp.bfloat16),
            pltpu.VMEM((SQ, D), jnp.bfloat16),
            pltpu.VMEM((SQ, D), jnp.bfloat16),
            pltpu.VMEM((2, SQ, DH), jnp.float32),
            pltpu.VMEM((2, SQ, DH), jnp.float32),
            pltpu.VMEM((SQ, DH), jnp.bfloat16),
            pltpu.VMEM((SQ, DH), jnp.bfloat16),
            pltpu.VMEM((3, QB, KB), jnp.float32),
            pltpu.SemaphoreType.DMA((2, 2)),
            pltpu.SemaphoreType.DMA((8,)),
        ],
        compiler_params=pltpu.CompilerParams(
            collective_id=0, vmem_limit_bytes=56 * 1024 * 1024
        ),
    )(x_bf, wq_bf, wo_bf, K_ext, V_ext)


# device time: 191858 ns/iter; 1.0580x vs baseline; 1.0580x over previous
import functools

import jax
import jax.numpy as jnp
from jax import lax
from jax.experimental import pallas as pl
from jax.experimental.pallas import tpu as pltpu

N_DEV = 4
SQ = 2048
D = 1024
H_PER = 8
DH = 128
QB = 256
KB = 512
N_QB = SQ // QB
WINDOW = 128
SCALE = 0.08838834764831843


def kernel(x, Wq, K_ext, V_ext, Wo):
    x_bf = x.astype(jnp.bfloat16)
    wq_bf = Wq.astype(jnp.bfloat16)
    wo_bf = Wo.astype(jnp.bfloat16)

    def body(x_ref, wq_ref, wo_ref, k_hbm, v_hbm, out_ref,
             w_buf, q_buf, ctx_buf, k_stage, v_stage, kh_bf, vh_bf, bias_ref,
             kv_sems, ring_sems):
        my = lax.axis_index("i")
        left = (my + N_DEV - 1) % N_DEV
        right = (my + 1) % N_DEV

        barrier_sem = pltpu.get_barrier_semaphore()
        for nbr in (left, right):
            pl.semaphore_signal(
                barrier_sem, inc=1,
                device_id=(nbr,), device_id_type=pl.DeviceIdType.MESH,
            )
        pl.semaphore_wait(barrier_sem, 2)

        w_buf[0, :D, :] = wq_ref[...]
        w_buf[0, D:, :] = wo_ref[...]

        for idx in range(3):
            off = idx * WINDOW
            qi = off + lax.broadcasted_iota(jnp.int32, (QB, KB), 0)
            ki = lax.broadcasted_iota(jnp.int32, (QB, KB), 1)
            bias_ref[idx] = jnp.where(
                jnp.abs(qi - ki) <= WINDOW,
                jnp.float32(0.0), jnp.float32(-1e9),
            )

        def kv_copies(g, slot):
            ck = pltpu.make_async_copy(
                k_hbm.at[my, :, g, :], k_stage.at[slot], kv_sems.at[slot, 0]
            )
            cv = pltpu.make_async_copy(
                v_hbm.at[my, :, g, :], v_stage.at[slot], kv_sems.at[slot, 1]
            )
            return ck, cv

        def compute_stage(slot, origin, first):
            g0 = origin * H_PER

            cops = {0: kv_copies(g0, 0)}
            cops[0][0].start()
            cops[0][1].start()

            q = lax.dot_general(
                x_ref[0], w_buf[slot, :D, :],
                (((1,), (0,)), ((), ())),
                preferred_element_type=jnp.float32,
            )
            q_buf[...] = (q * SCALE).astype(jnp.bfloat16)

            for h in range(H_PER):
                kv_slot = h % 2
                if h + 1 < H_PER:
                    cops[h + 1] = kv_copies(g0 + h + 1, (h + 1) % 2)
                    cops[h + 1][0].start()
                    cops[h + 1][1].start()
                cops[h][0].wait()
                cops[h][1].wait()
                kh_bf[...] = k_stage[kv_slot].astype(jnp.bfloat16)
                vh_bf[...] = v_stage[kv_slot].astype(jnp.bfloat16)

                def attn_block(q0, s, bias, h=h):
                    q_blk = q_buf[pl.ds(q0, QB), h * DH:(h + 1) * DH]
                    k_blk = kh_bf[pl.ds(s, KB), :]
                    v_blk = vh_bf[pl.ds(s, KB), :]
                    sc = lax.dot_general(
                        q_blk, k_blk, (((1,), (1,)), ((), ())),
                        preferred_element_type=jnp.float32,
                    )
                    w = jnp.exp(sc + bias)
                    denom = jnp.sum(w, axis=1, keepdims=True)
                    wb = w.astype(jnp.bfloat16)
                    ctx = lax.dot_general(
                        wb, v_blk, (((1,), (0,)), ((), ())),
                        preferred_element_type=jnp.float32,
                    )
                    ctx_buf[pl.ds(q0, QB), h * DH:(h + 1) * DH] = (
                        (ctx * (1.0 / denom)).astype(jnp.bfloat16)
                    )

                attn_block(0, 0, bias_ref[0])

                def qb_body(qb, _):
                    q0 = pl.multiple_of(qb * QB, QB)
                    s = pl.multiple_of(q0 - WINDOW, WINDOW)
                    attn_block(q0, s, bias_ref[1])
                    return 0

                lax.fori_loop(1, N_QB - 1, qb_body, 0)
                attn_block(SQ - QB, SQ - KB, bias_ref[2])

            partial = lax.dot_general(
                ctx_buf[...], w_buf[slot, D:, :],
                (((1,), (0,)), ((), ())),
                preferred_element_type=jnp.float32,
            )
            if first:
                out_ref[0] = partial
            else:
                out_ref[0] = out_ref[0] + partial

        def ring_copy(src, dst, sem_i, target):
            return pltpu.make_async_remote_copy(
                src_ref=src, dst_ref=dst,
                send_sem=ring_sems.at[sem_i], recv_sem=ring_sems.at[sem_i + 1],
                device_id=(target,), device_id_type=pl.DeviceIdType.MESH,
            )

        a_r = ring_copy(w_buf.at[0], w_buf.at[1], 0, right)
        a_l = ring_copy(w_buf.at[0], w_buf.at[2], 2, left)
        a_r.start()
        compute_stage(0, my, True)
        a_l.start()
        a_r.wait()
        b_r = ring_copy(
            w_buf.at[1, pl.ds(0, D)], w_buf.at[3, pl.ds(0, D)], 4, right
        )
        b_r.start()
        compute_stage(1, (my + N_DEV - 1) % N_DEV, False)
        a_l.wait()
        b_l = ring_copy(
            w_buf.at[2, pl.ds(D, D)], w_buf.at[3, pl.ds(D, D)], 6, left
        )
        b_l.start()
        compute_stage(2, (my + 1) % N_DEV, False)
        b_r.wait()
        b_l.wait()
        compute_stage(3, (my + 2) % N_DEV, False)

        @functools.partial(
            pl.run_scoped, second_barrier=pltpu.SemaphoreType.REGULAR
        )
        def _(second_barrier):
            for nbr in (left, right):
                pl.semaphore_signal(
                    second_barrier, inc=1,
                    device_id=(nbr,), device_id_type=pl.DeviceIdType.MESH,
                )
            pl.semaphore_wait(second_barrier, 2)

    return pl.pallas_call(
        body,
        out_shape=jax.ShapeDtypeStruct((1, SQ, D), jnp.float32),
        in_specs=[
            pl.BlockSpec(memory_space=pltpu.VMEM),
            pl.BlockSpec(memory_space=pltpu.VMEM),
            pl.BlockSpec(memory_space=pltpu.VMEM),
            pl.BlockSpec(memory_space=pl.ANY),
            pl.BlockSpec(memory_space=pl.ANY),
        ],
        out_specs=pl.BlockSpec(memory_space=pltpu.VMEM),
        scratch_shapes=[
            pltpu.VMEM((N_DEV, 2 * D, D), jnp.bfloat16),
            pltpu.VMEM((SQ, D), jnp.bfloat16),
            pltpu.VMEM((SQ, D), jnp.bfloat16),
            pltpu.VMEM((2, SQ, DH), jnp.float32),
            pltpu.VMEM((2, SQ, DH), jnp.float32),
            pltpu.VMEM((SQ, DH), jnp.bfloat16),
            pltpu.VMEM((SQ, DH), jnp.bfloat16),
            pltpu.VMEM((3, QB, KB), jnp.float32),
            pltpu.SemaphoreType.DMA((2, 2)),
            pltpu.SemaphoreType.DMA((8,)),
        ],
        compiler_params=pltpu.CompilerParams(
            collective_id=0, vmem_limit_bytes=56 * 1024 * 1024
        ),
    )(x_bf, wq_bf, wo_bf, K_ext, V_ext)


# device time: 191567 ns/iter; 1.0596x vs baseline; 1.0015x over previous
import functools

import jax
import jax.numpy as jnp
from jax import lax
from jax.experimental import pallas as pl
from jax.experimental.pallas import tpu as pltpu

N_DEV = 4
SQ = 2048
D = 1024
H_PER = 8
DH = 128
QB = 256
KB = 512
N_QB = SQ // QB
WINDOW = 128
SCALE = 0.08838834764831843


def kernel(x, Wq, K_ext, V_ext, Wo):
    x_bf = x.astype(jnp.bfloat16)
    wq_bf = Wq.astype(jnp.bfloat16)
    wo_bf = Wo.astype(jnp.bfloat16)

    def body(x_ref, wq_ref, wo_ref, k_hbm, v_hbm, out_ref,
             w_buf, q_buf, ctx_buf, k_stage, v_stage, kh_bf, vh_bf, bias_ref,
             kv_sems, ring_sems):
        my = lax.axis_index("i")
        left = (my + N_DEV - 1) % N_DEV
        right = (my + 1) % N_DEV

        barrier_sem = pltpu.get_barrier_semaphore()
        for nbr in (left, right):
            pl.semaphore_signal(
                barrier_sem, inc=1,
                device_id=(nbr,), device_id_type=pl.DeviceIdType.MESH,
            )
        pl.semaphore_wait(barrier_sem, 2)

        a_r_wq = pltpu.make_async_remote_copy(
            src_ref=wq_ref, dst_ref=w_buf.at[0, pl.ds(0, D)],
            send_sem=ring_sems.at[0], recv_sem=ring_sems.at[1],
            device_id=(right,), device_id_type=pl.DeviceIdType.MESH,
        )
        a_r_wo = pltpu.make_async_remote_copy(
            src_ref=wo_ref, dst_ref=w_buf.at[0, pl.ds(D, D)],
            send_sem=ring_sems.at[4], recv_sem=ring_sems.at[5],
            device_id=(right,), device_id_type=pl.DeviceIdType.MESH,
        )
        a_r_wq.start()
        a_r_wo.start()

        for idx in range(3):
            off = idx * WINDOW
            qi = off + lax.broadcasted_iota(jnp.int32, (QB, KB), 0)
            ki = lax.broadcasted_iota(jnp.int32, (QB, KB), 1)
            bias_ref[idx] = jnp.where(
                jnp.abs(qi - ki) <= WINDOW,
                jnp.float32(0.0), jnp.float32(-1e9),
            )

        def kv_copies(g, slot):
            ck = pltpu.make_async_copy(
                k_hbm.at[my, :, g, :], k_stage.at[slot], kv_sems.at[slot, 0]
            )
            cv = pltpu.make_async_copy(
                v_hbm.at[my, :, g, :], v_stage.at[slot], kv_sems.at[slot, 1]
            )
            return ck, cv

        def compute_stage(wq_mat, wo_mat, origin, first):
            g0 = origin * H_PER

            cops = {0: kv_copies(g0, 0)}
            cops[0][0].start()
            cops[0][1].start()

            q = lax.dot_general(
                x_ref[0], wq_mat,
                (((1,), (0,)), ((), ())),
                preferred_element_type=jnp.float32,
            )
            q_buf[...] = (q * SCALE).astype(jnp.bfloat16)

            for h in range(H_PER):
                kv_slot = h % 2
                if h + 1 < H_PER:
                    cops[h + 1] = kv_copies(g0 + h + 1, (h + 1) % 2)
                    cops[h + 1][0].start()
                    cops[h + 1][1].start()
                cops[h][0].wait()
                cops[h][1].wait()
                kh_bf[...] = k_stage[kv_slot].astype(jnp.bfloat16)
                vh_bf[...] = v_stage[kv_slot].astype(jnp.bfloat16)

                def attn_block(q0, s, bias, h=h):
                    q_blk = q_buf[pl.ds(q0, QB), h * DH:(h + 1) * DH]
                    k_blk = kh_bf[pl.ds(s, KB), :]
                    v_blk = vh_bf[pl.ds(s, KB), :]
                    sc = lax.dot_general(
                        q_blk, k_blk, (((1,), (1,)), ((), ())),
                        preferred_element_type=jnp.float32,
                    )
                    w = jnp.exp(sc + bias)
                    denom = jnp.sum(w, axis=1, keepdims=True)
                    wb = w.astype(jnp.bfloat16)
                    ctx = lax.dot_general(
                        wb, v_blk, (((1,), (0,)), ((), ())),
                        preferred_element_type=jnp.float32,
                    )
                    ctx_buf[pl.ds(q0, QB), h * DH:(h + 1) * DH] = (
                        (ctx * (1.0 / denom)).astype(jnp.bfloat16)
                    )

                attn_block(0, 0, bias_ref[0])

                def qb_body(qb, _):
                    q0 = pl.multiple_of(qb * QB, QB)
                    s = pl.multiple_of(q0 - WINDOW, WINDOW)
                    attn_block(q0, s, bias_ref[1])
                    return 0

                lax.fori_loop(1, N_QB - 1, qb_body, 0)
                attn_block(SQ - QB, SQ - KB, bias_ref[2])

            partial = lax.dot_general(
                ctx_buf[...], wo_mat,
                (((1,), (0,)), ((), ())),
                preferred_element_type=jnp.float32,
            )
            if first:
                out_ref[0] = partial
            else:
                out_ref[0] = out_ref[0] + partial

        def ring_copy(src, dst, sem_i, target):
            return pltpu.make_async_remote_copy(
                src_ref=src, dst_ref=dst,
                send_sem=ring_sems.at[sem_i], recv_sem=ring_sems.at[sem_i + 1],
                device_id=(target,), device_id_type=pl.DeviceIdType.MESH,
            )

        compute_stage(wq_ref[...], wo_ref[...], my, True)
        a_l_wq = ring_copy(wq_ref, w_buf.at[1, pl.ds(0, D)], 2, left)
        a_l_wo = ring_copy(wo_ref, w_buf.at[1, pl.ds(D, D)], 6, left)
        a_l_wq.start()
        a_l_wo.start()
        a_r_wq.wait()
        a_r_wo.wait()
        b_r = ring_copy(
            w_buf.at[0, pl.ds(0, D)], w_buf.at[2, pl.ds(0, D)], 8, right
        )
        b_r.start()
        compute_stage(
            w_buf[0, :D, :], w_buf[0, D:, :], (my + N_DEV - 1) % N_DEV, False
        )
        a_l_wq.wait()
        a_l_wo.wait()
        b_l = ring_copy(
            w_buf.at[1, pl.ds(D, D)], w_buf.at[2, pl.ds(D, D)], 10, left
        )
        b_l.start()
        compute_stage(
            w_buf[1, :D, :], w_buf[1, D:, :], (my + 1) % N_DEV, False
        )
        b_r.wait()
        b_l.wait()
        compute_stage(
            w_buf[2, :D, :], w_buf[2, D:, :], (my + 2) % N_DEV, False
        )

        @functools.partial(
            pl.run_scoped, second_barrier=pltpu.SemaphoreType.REGULAR
        )
        def _(second_barrier):
            for nbr in (left, right):
                pl.semaphore_signal(
                    second_barrier, inc=1,
                    device_id=(nbr,), device_id_type=pl.DeviceIdType.MESH,
                )
            pl.semaphore_wait(second_barrier, 2)

    return pl.pallas_call(
        body,
        out_shape=jax.ShapeDtypeStruct((1, SQ, D), jnp.float32),
        in_specs=[
            pl.BlockSpec(memory_space=pltpu.VMEM),
            pl.BlockSpec(memory_space=pltpu.VMEM),
            pl.BlockSpec(memory_space=pltpu.VMEM),
            pl.BlockSpec(memory_space=pl.ANY),
            pl.BlockSpec(memory_space=pl.ANY),
        ],
        out_specs=pl.BlockSpec(memory_space=pltpu.VMEM),
        scratch_shapes=[
            pltpu.VMEM((3, 2 * D, D), jnp.bfloat16),
            pltpu.VMEM((SQ, D), jnp.bfloat16),
            pltpu.VMEM((SQ, D), jnp.bfloat16),
            pltpu.VMEM((2, SQ, DH), jnp.float32),
            pltpu.VMEM((2, SQ, DH), jnp.float32),
            pltpu.VMEM((SQ, DH), jnp.bfloat16),
            pltpu.VMEM((SQ, DH), jnp.bfloat16),
            pltpu.VMEM((3, QB, KB), jnp.float32),
            pltpu.SemaphoreType.DMA((2, 2)),
            pltpu.SemaphoreType.DMA((12,)),
        ],
        compiler_params=pltpu.CompilerParams(
            collective_id=0, vmem_limit_bytes=56 * 1024 * 1024
        ),
    )(x_bf, wq_bf, wo_bf, K_ext, V_ext)
